# Initial kernel scaffold; baseline (speedup 1.0000x reference)
import jax
import jax.numpy as jnp
from jax.experimental import pallas as pl


def kernel(atomic_numbers, edge_distance, edge_index, wigner_inv, source_embedding, target_embedding, W1, b1, W2, b2):
    raise NotImplementedError("write your pallas kernel here")


# speedup vs baseline: 4.0190x; 4.0190x over previous
import functools

import jax
import jax.numpy as jnp
from jax.experimental import pallas as pl
from jax.experimental.pallas import tpu as pltpu

N_NODES = 10000
N_EDGES = 160000
NUM_RBF = 64
EDGE_CH = 64
SPHERE_CH = 64
NUM_COEFF = 16
M0_COEFF = 4
MAX_ELEM = 90
ELEM_PAD = 96
HIDDEN = 128
RESCALE = 16.0

EB = 400
NB = N_EDGES // EB


def _tc_body(ed_ref, an_s_ref, an_d_ref, wig_ref, dst_ref,
             S_ref, T_ref, W1_ref, b1_ref, W2_ref, b2_ref,
             out_ref, cs_ref):
    i = pl.program_id(0)

    @pl.when(i == 0)
    def _init():
        out_ref[...] = jnp.zeros_like(out_ref)

    ed = ed_ref[0]
    an_s = an_s_ref[0]
    an_d = an_d_ref[0]
    iot = jax.lax.broadcasted_iota(jnp.int32, (EB, ELEM_PAD), 1)
    oh_s = (an_s == iot).astype(jnp.float32)
    oh_d = (an_d == iot).astype(jnp.float32)
    x_src = jnp.dot(oh_s, S_ref[...], preferred_element_type=jnp.float32)
    x_dst = jnp.dot(oh_d, T_ref[...], preferred_element_type=jnp.float32)

    h = (jnp.dot(ed, W1_ref[0:NUM_RBF], preferred_element_type=jnp.float32)
         + jnp.dot(x_src, W1_ref[NUM_RBF:NUM_RBF + EDGE_CH],
                   preferred_element_type=jnp.float32)
         + jnp.dot(x_dst, W1_ref[NUM_RBF + EDGE_CH:],
                   preferred_element_type=jnp.float32)
         + b1_ref[...])
    h = jnp.maximum(h, 0.0)

    w = wig_ref[0]
    acc = None
    for j in range(M0_COEFF):
        m0j = (jnp.dot(h, W2_ref[:, SPHERE_CH * j:SPHERE_CH * (j + 1)],
                       preferred_element_type=jnp.float32)
               + b2_ref[:, SPHERE_CH * j:SPHERE_CH * (j + 1)]) * (1.0 / RESCALE)
        term = w[:, :, j][..., None] * m0j[:, None, :]
        acc = term if acc is None else acc + term
    cs_ref[...] = jnp.concatenate([acc[:, 0:8, :], acc[:, 8:16, :]], axis=2)

    def body(e, carry):
        d = dst_ref[0, 0, e]
        out_ref[d] = out_ref[d] + cs_ref[e]
        return carry

    jax.lax.fori_loop(0, EB, body, 0)


def kernel(atomic_numbers, edge_distance, edge_index, wigner_inv,
           source_embedding, target_embedding, W1, b1, W2, b2):
    src = edge_index[0]
    dst = edge_index[1]
    an_src = jnp.take(atomic_numbers, src, axis=0).astype(jnp.int32)
    an_dst = jnp.take(atomic_numbers, dst, axis=0).astype(jnp.int32)

    ed = edge_distance.reshape(NB, EB, NUM_RBF)
    an_s = an_src.reshape(NB, EB, 1)
    an_d = an_dst.reshape(NB, EB, 1)
    wig = wigner_inv.reshape(NB, EB, NUM_COEFF, M0_COEFF)
    dst2d = dst.astype(jnp.int32).reshape(NB, 1, EB)
    S = jnp.zeros((ELEM_PAD, EDGE_CH), jnp.float32).at[:MAX_ELEM].set(source_embedding)
    T = jnp.zeros((ELEM_PAD, EDGE_CH), jnp.float32).at[:MAX_ELEM].set(target_embedding)
    b1r = b1.reshape(1, HIDDEN)
    b2r = b2.reshape(1, M0_COEFF * SPHERE_CH)

    in_specs = [
            pl.BlockSpec((1, EB, NUM_RBF), lambda i: (i, 0, 0)),
            pl.BlockSpec((1, EB, 1), lambda i: (i, 0, 0)),
            pl.BlockSpec((1, EB, 1), lambda i: (i, 0, 0)),
            pl.BlockSpec((1, EB, NUM_COEFF, M0_COEFF), lambda i: (i, 0, 0, 0)),
            pl.BlockSpec((1, 1, EB), lambda i: (i, 0, 0), memory_space=pltpu.SMEM),
            pl.BlockSpec((ELEM_PAD, EDGE_CH), lambda i: (0, 0)),
            pl.BlockSpec((ELEM_PAD, EDGE_CH), lambda i: (0, 0)),
            pl.BlockSpec((NUM_RBF + 2 * EDGE_CH, HIDDEN), lambda i: (0, 0)),
            pl.BlockSpec((1, HIDDEN), lambda i: (0, 0)),
            pl.BlockSpec((HIDDEN, M0_COEFF * SPHERE_CH), lambda i: (0, 0)),
            pl.BlockSpec((1, M0_COEFF * SPHERE_CH), lambda i: (0, 0)),
    ]

    out = pl.pallas_call(
        _tc_body,
        grid=(NB,),
        in_specs=in_specs,
        out_specs=pl.BlockSpec((N_NODES, 8, 2 * SPHERE_CH),
                               lambda i: (0, 0, 0)),
        out_shape=jax.ShapeDtypeStruct((N_NODES, 8, 2 * SPHERE_CH),
                                       jnp.float32),
        scratch_shapes=[pltpu.VMEM((EB, 8, 2 * SPHERE_CH), jnp.float32)],
        compiler_params=pltpu.CompilerParams(
            dimension_semantics=("arbitrary",),
        ),
    )(ed, an_s, an_d, wig, dst2d, S, T, W1, b1r, W2, b2r)
    return jnp.concatenate([out[:, :, :SPHERE_CH], out[:, :, SPHERE_CH:]], axis=1)


# speedup vs baseline: 4.5761x; 1.1386x over previous
import functools

import jax
import jax.numpy as jnp
from jax.experimental import pallas as pl
from jax.experimental.pallas import tpu as pltpu

N_NODES = 10000
N_EDGES = 160000
NUM_RBF = 64
EDGE_CH = 64
SPHERE_CH = 64
NUM_COEFF = 16
M0_COEFF = 4
MAX_ELEM = 90
ELEM_PAD = 96
HIDDEN = 128
RESCALE = 16.0

EB = 400
NB = N_EDGES // EB


def _tc_body(ed_ref, an_s_ref, an_d_ref, wig_ref, dst_ref,
             S_ref, T_ref, W1_ref, b1_ref, W2_ref, b2_ref,
             out_ref, cs_ref):
    i = pl.program_id(0)

    @pl.when(i == 0)
    def _init():
        out_ref[...] = jnp.zeros_like(out_ref)

    ed = ed_ref[0]
    an_s = an_s_ref[0]
    an_d = an_d_ref[0]
    iot = jax.lax.broadcasted_iota(jnp.int32, (EB, ELEM_PAD), 1)
    oh_s = (an_s == iot).astype(jnp.float32)
    oh_d = (an_d == iot).astype(jnp.float32)
    x_src = jnp.dot(oh_s, S_ref[...], preferred_element_type=jnp.float32)
    x_dst = jnp.dot(oh_d, T_ref[...], preferred_element_type=jnp.float32)

    h = (jnp.dot(ed, W1_ref[0:NUM_RBF], preferred_element_type=jnp.float32)
         + jnp.dot(x_src, W1_ref[NUM_RBF:NUM_RBF + EDGE_CH],
                   preferred_element_type=jnp.float32)
         + jnp.dot(x_dst, W1_ref[NUM_RBF + EDGE_CH:],
                   preferred_element_type=jnp.float32)
         + b1_ref[...])
    h = jnp.maximum(h, 0.0)

    w = wig_ref[0]
    acc = None
    for j in range(M0_COEFF):
        m0j = (jnp.dot(h, W2_ref[:, SPHERE_CH * j:SPHERE_CH * (j + 1)],
                       preferred_element_type=jnp.float32)
               + b2_ref[:, SPHERE_CH * j:SPHERE_CH * (j + 1)]) * (1.0 / RESCALE)
        term = w[:, j, :][..., None] * m0j[:, None, :]
        acc = term if acc is None else acc + term
    cs_ref[...] = jnp.concatenate([acc[:, 0:8, :], acc[:, 8:16, :]], axis=2)

    def body(e, carry):
        d = dst_ref[0, 0, e]
        out_ref[d] = out_ref[d] + cs_ref[e]
        return carry

    jax.lax.fori_loop(0, EB, body, 0)


def kernel(atomic_numbers, edge_distance, edge_index, wigner_inv,
           source_embedding, target_embedding, W1, b1, W2, b2):
    src = edge_index[0]
    dst = edge_index[1]
    an_src = jnp.take(atomic_numbers, src, axis=0).astype(jnp.int32)
    an_dst = jnp.take(atomic_numbers, dst, axis=0).astype(jnp.int32)

    ed = edge_distance.reshape(NB, EB, NUM_RBF)
    an_s = an_src.reshape(NB, EB, 1)
    an_d = an_dst.reshape(NB, EB, 1)
    wig = wigner_inv.transpose(0, 2, 1).reshape(NB, EB, M0_COEFF, NUM_COEFF)
    dst2d = dst.astype(jnp.int32).reshape(NB, 1, EB)
    S = jnp.zeros((ELEM_PAD, EDGE_CH), jnp.float32).at[:MAX_ELEM].set(source_embedding)
    T = jnp.zeros((ELEM_PAD, EDGE_CH), jnp.float32).at[:MAX_ELEM].set(target_embedding)
    b1r = b1.reshape(1, HIDDEN)
    b2r = b2.reshape(1, M0_COEFF * SPHERE_CH)

    in_specs = [
            pl.BlockSpec((1, EB, NUM_RBF), lambda i: (i, 0, 0)),
            pl.BlockSpec((1, EB, 1), lambda i: (i, 0, 0)),
            pl.BlockSpec((1, EB, 1), lambda i: (i, 0, 0)),
            pl.BlockSpec((1, EB, M0_COEFF, NUM_COEFF), lambda i: (i, 0, 0, 0)),
            pl.BlockSpec((1, 1, EB), lambda i: (i, 0, 0), memory_space=pltpu.SMEM),
            pl.BlockSpec((ELEM_PAD, EDGE_CH), lambda i: (0, 0)),
            pl.BlockSpec((ELEM_PAD, EDGE_CH), lambda i: (0, 0)),
            pl.BlockSpec((NUM_RBF + 2 * EDGE_CH, HIDDEN), lambda i: (0, 0)),
            pl.BlockSpec((1, HIDDEN), lambda i: (0, 0)),
            pl.BlockSpec((HIDDEN, M0_COEFF * SPHERE_CH), lambda i: (0, 0)),
            pl.BlockSpec((1, M0_COEFF * SPHERE_CH), lambda i: (0, 0)),
    ]

    out = pl.pallas_call(
        _tc_body,
        grid=(NB,),
        in_specs=in_specs,
        out_specs=pl.BlockSpec((N_NODES, 8, 2 * SPHERE_CH),
                               lambda i: (0, 0, 0)),
        out_shape=jax.ShapeDtypeStruct((N_NODES, 8, 2 * SPHERE_CH),
                                       jnp.float32),
        scratch_shapes=[pltpu.VMEM((EB, 8, 2 * SPHERE_CH), jnp.float32)],
        compiler_params=pltpu.CompilerParams(
            dimension_semantics=("arbitrary",),
        ),
    )(ed, an_s, an_d, wig, dst2d, S, T, W1, b1r, W2, b2r)
    return jnp.concatenate([out[:, :, :SPHERE_CH], out[:, :, SPHERE_CH:]], axis=1)


# speedup vs baseline: 5.2285x; 1.1426x over previous
import functools

import jax
import jax.numpy as jnp
from jax.experimental import pallas as pl
from jax.experimental.pallas import tpu as pltpu

N_NODES = 10000
N_EDGES = 160000
NUM_RBF = 64
EDGE_CH = 64
SPHERE_CH = 64
NUM_COEFF = 16
M0_COEFF = 4
MAX_ELEM = 90
ELEM_PAD = 96
HIDDEN = 128
RESCALE = 16.0

EB = 400
NB = N_EDGES // EB


def _tc_body(ed_ref, an_s_ref, an_d_ref, wig_ref, dst_ref,
             S_ref, T_ref, W1_ref, b1_ref, W2_ref, b2_ref,
             out_ref, cs_ref):
    i = pl.program_id(0)

    @pl.when(i == 0)
    def _init():
        out_ref[...] = jnp.zeros_like(out_ref)

    ed = ed_ref[0]
    an_s = an_s_ref[0]
    an_d = an_d_ref[0]
    iot = jax.lax.broadcasted_iota(jnp.int32, (EB, ELEM_PAD), 1)
    oh_s = (an_s == iot).astype(jnp.float32)
    oh_d = (an_d == iot).astype(jnp.float32)
    x_src = jnp.dot(oh_s, S_ref[...], preferred_element_type=jnp.float32)
    x_dst = jnp.dot(oh_d, T_ref[...], preferred_element_type=jnp.float32)

    h = (jnp.dot(ed, W1_ref[0:NUM_RBF], preferred_element_type=jnp.float32)
         + jnp.dot(x_src, W1_ref[NUM_RBF:NUM_RBF + EDGE_CH],
                   preferred_element_type=jnp.float32)
         + jnp.dot(x_dst, W1_ref[NUM_RBF + EDGE_CH:],
                   preferred_element_type=jnp.float32)
         + b1_ref[...])
    h = jnp.maximum(h, 0.0)

    w = wig_ref[0]
    acc = None
    for j in range(M0_COEFF):
        m0j = (jnp.dot(h, W2_ref[:, SPHERE_CH * j:SPHERE_CH * (j + 1)],
                       preferred_element_type=jnp.float32)
               + b2_ref[:, SPHERE_CH * j:SPHERE_CH * (j + 1)]) * (1.0 / RESCALE)
        term = w[:, j, :][..., None] * m0j[:, None, :]
        acc = term if acc is None else acc + term
    cs_ref[...] = jnp.concatenate([acc[:, 0:8, :], acc[:, 8:16, :]], axis=2)

    def body(e, carry):
        d = dst_ref[0, 0, e]
        out_ref[d] = out_ref[d] + cs_ref[e]
        return carry

    jax.lax.fori_loop(0, EB, body, 0, unroll=8)


def kernel(atomic_numbers, edge_distance, edge_index, wigner_inv,
           source_embedding, target_embedding, W1, b1, W2, b2):
    src = edge_index[0]
    dst = edge_index[1]
    an_src = jnp.take(atomic_numbers, src, axis=0).astype(jnp.int32)
    an_dst = jnp.take(atomic_numbers, dst, axis=0).astype(jnp.int32)

    ed = edge_distance.reshape(NB, EB, NUM_RBF)
    an_s = an_src.reshape(NB, EB, 1)
    an_d = an_dst.reshape(NB, EB, 1)
    wig = wigner_inv.transpose(0, 2, 1).reshape(NB, EB, M0_COEFF, NUM_COEFF)
    dst2d = dst.astype(jnp.int32).reshape(NB, 1, EB)
    S = jnp.zeros((ELEM_PAD, EDGE_CH), jnp.float32).at[:MAX_ELEM].set(source_embedding)
    T = jnp.zeros((ELEM_PAD, EDGE_CH), jnp.float32).at[:MAX_ELEM].set(target_embedding)
    b1r = b1.reshape(1, HIDDEN)
    b2r = b2.reshape(1, M0_COEFF * SPHERE_CH)

    in_specs = [
            pl.BlockSpec((1, EB, NUM_RBF), lambda i: (i, 0, 0)),
            pl.BlockSpec((1, EB, 1), lambda i: (i, 0, 0)),
            pl.BlockSpec((1, EB, 1), lambda i: (i, 0, 0)),
            pl.BlockSpec((1, EB, M0_COEFF, NUM_COEFF), lambda i: (i, 0, 0, 0)),
            pl.BlockSpec((1, 1, EB), lambda i: (i, 0, 0), memory_space=pltpu.SMEM),
            pl.BlockSpec((ELEM_PAD, EDGE_CH), lambda i: (0, 0)),
            pl.BlockSpec((ELEM_PAD, EDGE_CH), lambda i: (0, 0)),
            pl.BlockSpec((NUM_RBF + 2 * EDGE_CH, HIDDEN), lambda i: (0, 0)),
            pl.BlockSpec((1, HIDDEN), lambda i: (0, 0)),
            pl.BlockSpec((HIDDEN, M0_COEFF * SPHERE_CH), lambda i: (0, 0)),
            pl.BlockSpec((1, M0_COEFF * SPHERE_CH), lambda i: (0, 0)),
    ]

    out = pl.pallas_call(
        _tc_body,
        grid=(NB,),
        in_specs=in_specs,
        out_specs=pl.BlockSpec((N_NODES, 8, 2 * SPHERE_CH),
                               lambda i: (0, 0, 0)),
        out_shape=jax.ShapeDtypeStruct((N_NODES, 8, 2 * SPHERE_CH),
                                       jnp.float32),
        scratch_shapes=[pltpu.VMEM((EB, 8, 2 * SPHERE_CH), jnp.float32)],
        compiler_params=pltpu.CompilerParams(
            dimension_semantics=("arbitrary",),
        ),
    )(ed, an_s, an_d, wig, dst2d, S, T, W1, b1r, W2, b2r)
    return jnp.concatenate([out[:, :, :SPHERE_CH], out[:, :, SPHERE_CH:]], axis=1)


# speedup vs baseline: 6.2392x; 1.1933x over previous
import functools

import jax
import jax.numpy as jnp
import numpy as np
from jax.experimental import pallas as pl
from jax.experimental.pallas import tpu as pltpu

N_NODES = 10000
N_EDGES = 160000
NUM_RBF = 64
EDGE_CH = 64
SPHERE_CH = 64
NUM_COEFF = 16
M0_COEFF = 4
MAX_ELEM = 90
ELEM_PAD = 96
HIDDEN = 128
RESCALE = 16.0

EB = 400
NB = N_EDGES // EB


def _tc_body(ed_ref, an_s_ref, an_d_ref, wig_ref, dst_ref,
             S_ref, T_ref, W1_ref, b1_ref, W2_ref, b2_ref, E_ref,
             out_ref, cs_ref):
    i = pl.program_id(0)

    @pl.when(i == 0)
    def _init():
        out_ref[...] = jnp.zeros_like(out_ref)

    ed = ed_ref[0]
    an_s = an_s_ref[0]
    an_d = an_d_ref[0]
    iot = jax.lax.broadcasted_iota(jnp.int32, (EB, ELEM_PAD), 1)
    oh_s = (an_s == iot).astype(jnp.float32)
    oh_d = (an_d == iot).astype(jnp.float32)
    x_src = jnp.dot(oh_s, S_ref[...], preferred_element_type=jnp.float32)
    x_dst = jnp.dot(oh_d, T_ref[...], preferred_element_type=jnp.float32)

    h = (jnp.dot(ed, W1_ref[0:NUM_RBF], preferred_element_type=jnp.float32)
         + jnp.dot(x_src, W1_ref[NUM_RBF:NUM_RBF + EDGE_CH],
                   preferred_element_type=jnp.float32)
         + jnp.dot(x_dst, W1_ref[NUM_RBF + EDGE_CH:],
                   preferred_element_type=jnp.float32)
         + b1_ref[...])
    h = jnp.maximum(h, 0.0)

    w64 = wig_ref[0]
    m0 = (jnp.dot(h, W2_ref[...], preferred_element_type=jnp.float32)
          + b2_ref[...]) * (1.0 / RESCALE)
    m0dup = jnp.concatenate(
        [m0[:, SPHERE_CH * j:SPHERE_CH * (j + 1)] for j in range(M0_COEFF)
         for _ in range(2)], axis=1)
    for r in range(8):
        wexp = jnp.dot(w64, E_ref[:, 512 * r:512 * (r + 1)],
                       preferred_element_type=jnp.float32)
        p = wexp * m0dup
        cs_ref[:, r, :] = (p[:, 0:128] + p[:, 128:256]
                           + p[:, 256:384] + p[:, 384:512])

    def body(e, carry):
        d = dst_ref[0, 0, e]
        out_ref[d] = out_ref[d] + cs_ref[e]
        return carry

    jax.lax.fori_loop(0, EB, body, 0, unroll=8)


def kernel(atomic_numbers, edge_distance, edge_index, wigner_inv,
           source_embedding, target_embedding, W1, b1, W2, b2):
    src = edge_index[0]
    dst = edge_index[1]
    an_src = jnp.take(atomic_numbers, src, axis=0).astype(jnp.int32)
    an_dst = jnp.take(atomic_numbers, dst, axis=0).astype(jnp.int32)

    ed = edge_distance.reshape(NB, EB, NUM_RBF)
    an_s = an_src.reshape(NB, EB, 1)
    an_d = an_dst.reshape(NB, EB, 1)
    wig = wigner_inv.transpose(0, 2, 1).reshape(NB, EB, M0_COEFF * NUM_COEFF)
    q_j, q_i = np.divmod(np.arange(64), 16)
    u = np.arange(4096)
    u_r, u_rem = np.divmod(u, 512)
    u_j, u_rem2 = np.divmod(u_rem, 128)
    u_d = u_rem2 // 64
    e_mat = ((q_j[:, None] == u_j[None, :])
             & (q_i[:, None] == 2 * u_r[None, :] + u_d[None, :]))
    E_all = jnp.asarray(e_mat.astype(np.float32))
    dst2d = dst.astype(jnp.int32).reshape(NB, 1, EB)
    S = jnp.zeros((ELEM_PAD, EDGE_CH), jnp.float32).at[:MAX_ELEM].set(source_embedding)
    T = jnp.zeros((ELEM_PAD, EDGE_CH), jnp.float32).at[:MAX_ELEM].set(target_embedding)
    b1r = b1.reshape(1, HIDDEN)
    b2r = b2.reshape(1, M0_COEFF * SPHERE_CH)

    in_specs = [
            pl.BlockSpec((1, EB, NUM_RBF), lambda i: (i, 0, 0)),
            pl.BlockSpec((1, EB, 1), lambda i: (i, 0, 0)),
            pl.BlockSpec((1, EB, 1), lambda i: (i, 0, 0)),
            pl.BlockSpec((1, EB, M0_COEFF * NUM_COEFF), lambda i: (i, 0, 0)),
            pl.BlockSpec((1, 1, EB), lambda i: (i, 0, 0), memory_space=pltpu.SMEM),
            pl.BlockSpec((ELEM_PAD, EDGE_CH), lambda i: (0, 0)),
            pl.BlockSpec((ELEM_PAD, EDGE_CH), lambda i: (0, 0)),
            pl.BlockSpec((NUM_RBF + 2 * EDGE_CH, HIDDEN), lambda i: (0, 0)),
            pl.BlockSpec((1, HIDDEN), lambda i: (0, 0)),
            pl.BlockSpec((HIDDEN, M0_COEFF * SPHERE_CH), lambda i: (0, 0)),
            pl.BlockSpec((1, M0_COEFF * SPHERE_CH), lambda i: (0, 0)),
            pl.BlockSpec((64, 4096), lambda i: (0, 0)),
    ]

    out = pl.pallas_call(
        _tc_body,
        grid=(NB,),
        in_specs=in_specs,
        out_specs=pl.BlockSpec((N_NODES, 8, 2 * SPHERE_CH),
                               lambda i: (0, 0, 0)),
        out_shape=jax.ShapeDtypeStruct((N_NODES, 8, 2 * SPHERE_CH),
                                       jnp.float32),
        scratch_shapes=[pltpu.VMEM((EB, 8, 2 * SPHERE_CH), jnp.float32)],
        compiler_params=pltpu.CompilerParams(
            dimension_semantics=("arbitrary",),
        ),
    )(ed, an_s, an_d, wig, dst2d, S, T, W1, b1r, W2, b2r, E_all)
    return out.reshape(N_NODES, NUM_COEFF, SPHERE_CH)
